# Initial kernel scaffold; baseline (speedup 1.0000x reference)
import jax
import jax.numpy as jnp
from jax.experimental import pallas as pl


def kernel(x1, edge_index1, batch1, x2, edge_index2, batch2, rel, W1_0, b1_0, W2_0, b2_0, gamma_0, beta_0, W1_1, b1_1, W2_1, b2_1, gamma_1, beta_1, W1_2, b1_2, W2_2, b2_2, gamma_2, beta_2, kge_table, fcW1, fcb1, fcW2, fcb2):
    raise NotImplementedError("write your pallas kernel here")


# speedup vs baseline: 3.2205x; 3.2205x over previous
import functools

import jax
import jax.numpy as jnp
from jax import lax
from jax.experimental import pallas as pl
from jax.experimental.pallas import tpu as pltpu
from jax.experimental.pallas import tpu_sc as plsc

_N = 10000
_NP = 10240
_D = 128
_B = 512
_E = 320000
_NW = 32
_CH = 128
_S = (_E + _NW * _CH - 1) // (_NW * _CH)
_EPAD = _NW * _S * _CH
_RPW = _NP // 16
_ZR = 64
_HI = lax.Precision.HIGHEST


def _sc_agg_body(x_hbm, src_hbm, dst_hbm, out_hbm,
                 src_v, dst_v, rows_v, zero_v, agg_sh, sem):
    c = lax.axis_index("c")
    s = lax.axis_index("s")
    wid = s * 2 + c

    zeros16 = jnp.zeros((16,), jnp.float32)

    @pl.loop(0, _ZR)
    def _zero_fill(i):
        for j in range(_D // 16):
            zero_v[i, pl.ds(j * 16, 16)] = zeros16

    @pl.loop(0, _RPW // _ZR)
    def _zero_acc(k):
        pltpu.sync_copy(zero_v, agg_sh.at[pl.ds(s * _RPW + k * _ZR, _ZR)])

    plsc.subcore_barrier()

    pltpu.sync_copy(src_hbm.at[wid], src_v)
    pltpu.sync_copy(dst_hbm.at[wid], dst_v)

    @pl.loop(0, _S)
    def _edge_step(j):
        pltpu.async_copy(x_hbm.at[src_v.at[j]], rows_v, sem).wait()
        pltpu.sync_copy(rows_v, agg_sh.at[dst_v.at[j]], add=True)

    plsc.subcore_barrier()

    pltpu.sync_copy(agg_sh.at[pl.ds(s * _RPW, _RPW)],
                    out_hbm.at[c, pl.ds(s * _RPW, _RPW)])


@functools.cache
def _sc_agg_fn():
    mesh = plsc.VectorSubcoreMesh(core_axis_name="c", subcore_axis_name="s")
    return pl.kernel(
        _sc_agg_body,
        out_type=jax.ShapeDtypeStruct((2, _NP, _D), jnp.float32),
        mesh=mesh,
        scratch_types=[
            pltpu.VMEM((_S, _CH), jnp.int32),
            pltpu.VMEM((_S, _CH), jnp.int32),
            pltpu.VMEM((_CH, _D), jnp.float32),
            pltpu.VMEM((_ZR, _D), jnp.float32),
            pltpu.VMEM_SHARED((_NP, _D), jnp.float32),
            pltpu.SemaphoreType.DMA,
        ],
    )


def _mlp_body(x_ref, agg_ref, w1_ref, b1_ref, w2_ref, b2_ref,
              u_ref, sum_ref):
    i = pl.program_id(0)
    h = x_ref[...] + agg_ref[0] + agg_ref[1]
    t = jnp.maximum(lax.dot(h, w1_ref[...]) + b1_ref[...], 0.0)
    u = lax.dot(t, w2_ref[...]) + b2_ref[...]
    u_ref[...] = u

    @pl.when(i == 0)
    def _init():
        sum_ref[...] = jnp.zeros_like(sum_ref)

    rows = i * 512 + lax.broadcasted_iota(jnp.int32, (512, 1), 0)
    um = jnp.where(rows < _N, u, 0.0)
    sum_ref[...] += jnp.sum(um, axis=0, keepdims=True)


def _var_body(u_ref, sum_ref, vo_ref):
    i = pl.program_id(0)

    @pl.when(i == 0)
    def _init():
        vo_ref[...] = jnp.zeros_like(vo_ref)

    m = sum_ref[...] / _N
    rows = i * 512 + lax.broadcasted_iota(jnp.int32, (512, 1), 0)
    dd = jnp.where(rows < _N, u_ref[...] - m, 0.0)
    vo_ref[...] += jnp.sum(dd * dd, axis=0, keepdims=True)


def _var_call(u, sm):
    grid = _NP // 512
    return pl.pallas_call(
        _var_body,
        grid=(grid,),
        in_specs=[
            pl.BlockSpec((512, _D), lambda i: (i, 0)),
            pl.BlockSpec((1, _D), lambda i: (0, 0)),
        ],
        out_specs=pl.BlockSpec((1, _D), lambda i: (0, 0)),
        out_shape=jax.ShapeDtypeStruct((1, _D), jnp.float32),
    )(u, sm)


def _mlp_call(xp, agg, w1, b1, w2, b2):
    grid = _NP // 512
    return pl.pallas_call(
        _mlp_body,
        grid=(grid,),
        in_specs=[
            pl.BlockSpec((512, _D), lambda i: (i, 0)),
            pl.BlockSpec((2, 512, _D), lambda i: (0, i, 0)),
            pl.BlockSpec((_D, _D), lambda i: (0, 0)),
            pl.BlockSpec((1, _D), lambda i: (0, 0)),
            pl.BlockSpec((_D, _D), lambda i: (0, 0)),
            pl.BlockSpec((1, _D), lambda i: (0, 0)),
        ],
        out_specs=[
            pl.BlockSpec((512, _D), lambda i: (i, 0)),
            pl.BlockSpec((1, _D), lambda i: (0, 0)),
        ],
        out_shape=[
            jax.ShapeDtypeStruct((_NP, _D), jnp.float32),
            jax.ShapeDtypeStruct((1, _D), jnp.float32),
        ],
    )(xp, agg, w1, b1, w2, b2)


def _bn_body(u_ref, sum_ref, ssq_ref, g_ref, b_ref, o_ref):
    m = sum_ref[...] / _N
    v = ssq_ref[...] / _N
    o_ref[...] = jnp.maximum(
        (u_ref[...] - m) / jnp.sqrt(v + 1e-5) * g_ref[...] + b_ref[...], 0.0)


def _bn_call(u, sm, sq, g, b):
    grid = _NP // 512
    return pl.pallas_call(
        _bn_body,
        grid=(grid,),
        in_specs=[
            pl.BlockSpec((512, _D), lambda i: (i, 0)),
            pl.BlockSpec((1, _D), lambda i: (0, 0)),
            pl.BlockSpec((1, _D), lambda i: (0, 0)),
            pl.BlockSpec((1, _D), lambda i: (0, 0)),
            pl.BlockSpec((1, _D), lambda i: (0, 0)),
        ],
        out_specs=pl.BlockSpec((512, _D), lambda i: (i, 0)),
        out_shape=jax.ShapeDtypeStruct((_NP, _D), jnp.float32),
    )(u, sm, sq, g, b)


def _pool_body(x_ref, b3_ref, p_ref):
    i = pl.program_id(0)

    @pl.when(i == 0)
    def _init():
        p_ref[...] = jnp.zeros_like(p_ref)

    bids = b3_ref[0]
    ohT = (lax.broadcasted_iota(jnp.int32, (_B, 512), 0) == bids)
    p_ref[...] += lax.dot(ohT.astype(jnp.float32), x_ref[...], precision=_HI)


def _pool_call(xp, b3):
    grid = _NP // 512
    return pl.pallas_call(
        _pool_body,
        grid=(grid,),
        in_specs=[
            pl.BlockSpec((512, _D), lambda i: (i, 0)),
            pl.BlockSpec((1, 1, 512), lambda i: (i, 0, 0)),
        ],
        out_specs=pl.BlockSpec((_B, _D), lambda i: (0, 0)),
        out_shape=jax.ShapeDtypeStruct((_B, _D), jnp.float32),
    )(xp, b3)


def _head_body(p1_ref, p2_ref, relb_ref, kge_ref, w1_ref, b1_ref,
               w2_ref, b2_ref, o_ref):
    oh = (relb_ref[...] == lax.broadcasted_iota(jnp.int32, (_B, _D), 1))
    r = lax.dot(oh.astype(jnp.float32), kge_ref[...], precision=_HI)
    z = (lax.dot(p1_ref[...], w1_ref[0:_D])
         + lax.dot(p2_ref[...], w1_ref[_D:2 * _D])
         + lax.dot(r, w1_ref[2 * _D:3 * _D])
         + b1_ref[...])
    t = jnp.maximum(z, 0.0)
    o_ref[...] = lax.dot(t, w2_ref[...]) + b2_ref[...]


def _head_call(p1, p2, relb, kge_pad, fcw1, fcb1, fcw2p, fcb2p):
    return pl.pallas_call(
        _head_body,
        out_shape=jax.ShapeDtypeStruct((_B, _D), jnp.float32),
    )(p1, p2, relb, kge_pad, fcw1, fcb1, fcw2p, fcb2p)


def _prep_edges(ei):
    src = ei[0].astype(jnp.int32)
    dst = ei[1].astype(jnp.int32)
    pad = _EPAD - _E
    src = jnp.pad(src, (0, pad)).reshape(_NW, _S, _CH)
    dst = jnp.pad(dst, (0, pad), constant_values=_N).reshape(_NW, _S, _CH)
    return src, dst


def _prep_batch(b):
    bp = jnp.pad(b.astype(jnp.int32), (0, _NP - _N), constant_values=_B)
    return bp.reshape(_NP // 512, 1, 512)


def kernel(x1, edge_index1, batch1, x2, edge_index2, batch2, rel,
           W1_0, b1_0, W2_0, b2_0, gamma_0, beta_0,
           W1_1, b1_1, W2_1, b2_1, gamma_1, beta_1,
           W1_2, b1_2, W2_2, b2_2, gamma_2, beta_2,
           kge_table, fcW1, fcb1, fcW2, fcb2):
    layers = [
        (W1_0, b1_0.reshape(1, -1), W2_0, b2_0.reshape(1, -1),
         gamma_0.reshape(1, -1), beta_0.reshape(1, -1)),
        (W1_1, b1_1.reshape(1, -1), W2_1, b2_1.reshape(1, -1),
         gamma_1.reshape(1, -1), beta_1.reshape(1, -1)),
        (W1_2, b1_2.reshape(1, -1), W2_2, b2_2.reshape(1, -1),
         gamma_2.reshape(1, -1), beta_2.reshape(1, -1)),
    ]
    sc_agg = _sc_agg_fn()

    def gnn(x, ei):
        xp = jnp.pad(x, ((0, _NP - _N), (0, 0)))
        src3, dst3 = _prep_edges(ei)
        for (w1, b1, w2, b2, g, bt) in layers:
            agg = sc_agg(xp, src3, dst3)
            u, sm = _mlp_call(xp, agg, w1, b1, w2, b2)
            sq = _var_call(u, sm)
            xp = _bn_call(u, sm, sq, g, bt)
        return xp

    h1 = gnn(x1, edge_index1)
    h2 = gnn(x2, edge_index2)
    p1 = _pool_call(h1, _prep_batch(batch1))
    p2 = _pool_call(h2, _prep_batch(batch2))

    relb = jnp.broadcast_to(rel.astype(jnp.int32), (_B, _D))
    kge_pad = jnp.pad(kge_table, ((0, _D - kge_table.shape[0]), (0, 0)))
    fcw2p = jnp.pad(fcW2, ((0, 0), (0, _D - 1)))
    fcb2p = jnp.pad(fcb2, (0, _D - 1)).reshape(1, _D)
    o = _head_call(p1, p2, relb, kge_pad, fcW1, fcb1.reshape(1, -1),
                   fcw2p, fcb2p)
    return o[:, 0]
